# baseline (speedup 1.0000x reference)
import jax
import jax.numpy as jnp
from jax.experimental import pallas as pl

_NUM_EMB = 8192
_DIM = 64
_COMMIT = 0.25
_TB = 256


def _vq_body(z_ref, embT_ref, emb_ref, zq_ref, idx_ref, acc_ref):
    zb = z_ref[...]
    embT = embT_ref[...]
    z2 = jnp.sum(zb * zb, axis=1, keepdims=True)
    e2 = jnp.sum(embT * embT, axis=0, keepdims=True)
    mm = jnp.dot(zb, embT, preferred_element_type=jnp.float32)
    dist = (z2 + e2) - 2.0 * mm

    minval = jnp.min(dist, axis=1, keepdims=True)
    iota = jax.lax.broadcasted_iota(jnp.int32, dist.shape, 1)
    idx = jnp.min(jnp.where(dist == minval, iota, jnp.int32(2**30)), axis=1)

    oh = (iota == idx[:, None]).astype(jnp.float32)
    zq = jnp.dot(oh, emb_ref[...], preferred_element_type=jnp.float32)

    zq_ref[...] = zb + (zq - zb)
    idx_ref[...] = idx[:, None]

    @pl.when(pl.program_id(0) == 0)
    def _init():
        acc_ref[...] = jnp.zeros_like(acc_ref)

    diff = zq - zb
    acc_ref[...] += jnp.sum(diff * diff)


def kernel(z, emb):
    b, c, h, w = z.shape
    zt = jnp.transpose(z, (0, 2, 3, 1))
    z_flat = zt.reshape(-1, _DIM)
    n_tok = z_flat.shape[0]
    embT = emb.T

    grid = (n_tok // _TB,)
    zq_flat, idx_col, acc = pl.pallas_call(
        _vq_body,
        grid=grid,
        in_specs=[
            pl.BlockSpec((_TB, _DIM), lambda i: (i, 0)),
            pl.BlockSpec((_DIM, _NUM_EMB), lambda i: (0, 0)),
            pl.BlockSpec((_NUM_EMB, _DIM), lambda i: (0, 0)),
        ],
        out_specs=[
            pl.BlockSpec((_TB, _DIM), lambda i: (i, 0)),
            pl.BlockSpec((_TB, 1), lambda i: (i, 0)),
            pl.BlockSpec((1, 1), lambda i: (0, 0)),
        ],
        out_shape=[
            jax.ShapeDtypeStruct((n_tok, _DIM), jnp.float32),
            jax.ShapeDtypeStruct((n_tok, 1), jnp.int32),
            jax.ShapeDtypeStruct((1, 1), jnp.float32),
        ],
    )(z_flat, embT, emb)

    z_q_st = jnp.transpose(zq_flat.reshape(b, h, w, c), (0, 3, 1, 2))
    indices = idx_col.reshape(b, h, w)
    loss = acc[0, 0] / jnp.float32(z.size)
    vq_loss = loss + _COMMIT * loss
    return (z_q_st, vq_loss, loss, loss, indices)


# speedup vs baseline: 1.6797x; 1.6797x over previous
import functools

import jax
import jax.numpy as jnp
from jax import lax
from jax.experimental import pallas as pl
from jax.experimental.pallas import tpu as pltpu
from jax.experimental.pallas import tpu_sc as plsc

_NUM_EMB = 8192
_DIM = 64
_COMMIT = 0.25
_TB = 256
_N_TOK = 16384

_NC = 2
_NS = 16
_NW = _NC * _NS
_BPW = _N_TOK // _NW
_IDX_CHUNK = 128
_NCH = _BPW // _IDX_CHUNK


def _argmin_body(z_ref, embT_ref, idx_ref, acc_ref):
    zb = z_ref[...]
    embT = embT_ref[...]
    z2 = jnp.sum(zb * zb, axis=1, keepdims=True)
    e2 = jnp.sum(embT * embT, axis=0, keepdims=True)
    mm = jnp.dot(zb, embT, preferred_element_type=jnp.float32)
    dist = (z2 + e2) - 2.0 * mm

    minval = jnp.min(dist, axis=1, keepdims=True)
    iota = jax.lax.broadcasted_iota(jnp.int32, dist.shape, 1)
    idx = jnp.min(jnp.where(dist == minval, iota, jnp.int32(2**30)), axis=1)
    idx_ref[...] = idx[:, None]

    @pl.when(pl.program_id(0) == 0)
    def _init():
        acc_ref[...] = jnp.zeros_like(acc_ref)

    acc_ref[...] += jnp.sum(minval)


def _gather_body(table_hbm, idx_hbm, out_hbm, idx_v, rows_v, sem):
    wid = lax.axis_index("s") * _NC + lax.axis_index("c")
    base = wid * _BPW
    pltpu.sync_copy(idx_hbm.at[pl.ds(wid * _NCH, _NCH)], idx_v)
    for j in range(_NCH):
        pltpu.async_copy(table_hbm.at[idx_v.at[j]],
                         rows_v.at[pl.ds(j * _IDX_CHUNK, _IDX_CHUNK)], sem)
    for _ in range(_NCH):
        pltpu.make_async_copy(table_hbm.at[idx_v.at[0]],
                              rows_v.at[pl.ds(0, _IDX_CHUNK)], sem).wait()
    pltpu.sync_copy(rows_v, out_hbm.at[pl.ds(base, _BPW)])


_sc_gather = functools.partial(
    pl.kernel,
    out_type=jax.ShapeDtypeStruct((_N_TOK, _DIM), jnp.float32),
    mesh=plsc.VectorSubcoreMesh(core_axis_name="c", subcore_axis_name="s",
                                num_cores=_NC, num_subcores=_NS),
    scratch_types=[
        pltpu.VMEM((_NCH, _IDX_CHUNK), jnp.int32),
        pltpu.VMEM((_BPW, _DIM), jnp.float32),
        pltpu.SemaphoreType.DMA,
    ],
    compiler_params=pltpu.CompilerParams(use_tc_tiling_on_sc=False),
)(_gather_body)


def kernel(z, emb):
    b, c, h, w = z.shape
    zt = jnp.transpose(z, (0, 2, 3, 1))
    z_flat = zt.reshape(-1, _DIM)
    embT = emb.T

    idx_col, acc = pl.pallas_call(
        _argmin_body,
        grid=(_N_TOK // _TB,),
        in_specs=[
            pl.BlockSpec((_TB, _DIM), lambda i: (i, 0)),
            pl.BlockSpec((_DIM, _NUM_EMB), lambda i: (0, 0)),
        ],
        out_specs=[
            pl.BlockSpec((_TB, 1), lambda i: (i, 0)),
            pl.BlockSpec((1, 1), lambda i: (0, 0)),
        ],
        out_shape=[
            jax.ShapeDtypeStruct((_N_TOK, 1), jnp.int32),
            jax.ShapeDtypeStruct((1, 1), jnp.float32),
        ],
    )(z_flat, embT)

    idx2d = idx_col.reshape(_NW * _NCH, _IDX_CHUNK)
    zq_flat = _sc_gather(emb, idx2d)

    zq_t = zq_flat.reshape(b, h, w, c)
    z_q_st = jnp.transpose(zt + (zq_t - zt), (0, 3, 1, 2))
    indices = idx_col.reshape(b, h, w)
    loss = acc[0, 0] / jnp.float32(z.size)
    vq_loss = loss + _COMMIT * loss
    return (z_q_st, vq_loss, loss, loss, indices)
